# baseline (speedup 1.0000x reference)
import functools

import jax
import jax.numpy as jnp
from jax import lax
from jax.experimental import pallas as pl
from jax.experimental.pallas import tpu as pltpu
from jax.experimental.pallas import tpu_sc as plsc


N = 10000
E = 160000
F = 256
FH = F // 2
HD = 1024
BM = 1000

NC = 2
NS = 16
NP = 10112
RPT = NP // NS

K2 = 128
EPT = 10240
CH2 = EPT // K2

K1 = 128
EPT1 = 5120
CH1 = EPT1 // K1

_MESH = plsc.VectorSubcoreMesh(core_axis_name="c", subcore_axis_name="s")


def _deg_body(z128, ones_h, srcd, degp_out, deg_sh, ones_v, idx_v):
    c = lax.axis_index("c")
    s = lax.axis_index("s")
    sl = pl.ds(s * RPT, RPT)
    pltpu.sync_copy(z128.at[sl], deg_sh.at[sl])
    pltpu.sync_copy(ones_h, ones_v)
    pltpu.sync_copy(srcd.at[c, s], idx_v)
    plsc.subcore_barrier()

    def body(j, carry):
        pltpu.sync_copy(ones_v, deg_sh.at[idx_v.at[j]], add=True)
        return carry

    lax.fori_loop(0, CH1, body, 0)
    plsc.subcore_barrier()
    pltpu.sync_copy(deg_sh.at[sl], degp_out.at[c, sl])


_deg_call = pl.kernel(
    _deg_body,
    out_type=jax.ShapeDtypeStruct((NC, NP, FH), jnp.float32),
    mesh=_MESH,
    scratch_types=[
        pltpu.VMEM_SHARED((NP, FH), jnp.float32),
        pltpu.VMEM((K1, FH), jnp.float32),
        pltpu.VMEM((CH1, K1), jnp.int32),
    ],
)


def _scat_body(xs2, z128, srcf, dst3, u_out,
               u_sh, dst_v, idx_a, idx_b, rows0, rows1, sem_i, sem_g0, sem_g1):
    c = lax.axis_index("c")
    s = lax.axis_index("s")
    sl = pl.ds(s * RPT, RPT)
    pltpu.sync_copy(z128.at[sl], u_sh.at[sl])
    pltpu.sync_copy(dst3.at[s], dst_v)
    base = (c * NS + s) * ((CH2 + 4) * K2)
    plsc.subcore_barrier()

    pltpu.sync_copy(srcf.at[pl.ds(base, 2 * K2)], idx_a)
    pltpu.async_copy(srcf.at[pl.ds(base + 2 * K2, 2 * K2)], idx_b, sem_i)
    pltpu.async_copy(xs2.at[idx_a.at[pl.ds(0, K2)]], rows0, sem_g0)

    def halfpair(j, i_a, i_b):
        pltpu.async_copy(xs2.at[i_a.at[pl.ds(K2, K2)]], rows1, sem_g1)
        pltpu.make_async_copy(xs2.at[i_a.at[pl.ds(0, K2)]], rows0, sem_g0).wait()
        pltpu.sync_copy(rows0, u_sh.at[dst_v.at[j]], add=True)
        pltpu.make_async_copy(srcf.at[pl.ds(base, 2 * K2)], i_b, sem_i).wait()
        pltpu.async_copy(xs2.at[i_b.at[pl.ds(0, K2)]], rows0, sem_g0)
        pltpu.make_async_copy(xs2.at[i_a.at[pl.ds(K2, K2)]], rows1, sem_g1).wait()
        pltpu.sync_copy(rows1, u_sh.at[dst_v.at[j + 1]], add=True)
        pltpu.async_copy(srcf.at[pl.ds(base + (j + 4) * K2, 2 * K2)], i_a, sem_i)

    def q_body(q, carry):
        j = 4 * q
        halfpair(j, idx_a, idx_b)
        halfpair(j + 2, idx_b, idx_a)
        return carry

    lax.fori_loop(0, CH2 // 4, q_body, 0)
    pltpu.make_async_copy(srcf.at[pl.ds(base, 2 * K2)], idx_b, sem_i).wait()
    pltpu.make_async_copy(xs2.at[idx_a.at[pl.ds(0, K2)]], rows0, sem_g0).wait()
    plsc.subcore_barrier()
    pltpu.sync_copy(u_sh.at[sl], u_out.at[c, sl])


_scat_call = pl.kernel(
    _scat_body,
    out_type=jax.ShapeDtypeStruct((NC, NP, FH), jnp.float32),
    mesh=_MESH,
    scratch_types=[
        pltpu.VMEM_SHARED((NP, FH), jnp.float32),
        pltpu.VMEM((CH2, K2), jnp.int32),
        pltpu.VMEM((2 * K2,), jnp.int32),
        pltpu.VMEM((2 * K2,), jnp.int32),
        pltpu.VMEM((K2, FH), jnp.float32),
        pltpu.VMEM((K2, FH), jnp.float32),
        pltpu.SemaphoreType.DMA,
        pltpu.SemaphoreType.DMA,
        pltpu.SemaphoreType.DMA,
    ],
)


def _xs_body(x_ref, degp_ref, xs2_ref):
    d = degp_ref[0, :, 0:1] + degp_ref[1, :, 0:1]
    dis = jnp.where(d > 0, lax.rsqrt(jnp.maximum(d, 1.0)), 0.0)
    xs2_ref[...] = (-dis) * x_ref[...]


def _xs_prep(x, degp):
    nb = N // BM
    return pl.pallas_call(
        _xs_body,
        grid=(2, nb),
        in_specs=[
            pl.BlockSpec((BM, FH), lambda h, i: (i, h)),
            pl.BlockSpec((NC, BM, FH), lambda h, i: (0, i, 0)),
        ],
        out_specs=pl.BlockSpec((BM, FH), lambda h, i: (h * (N // BM) + i, 0)),
        out_shape=jax.ShapeDtypeStruct((2 * N, FH), jnp.float32),
    )(x, degp)


def _xgate_body(xb, wxz0, wxh0, gx_out):
    f32 = jnp.float32
    x = xb[...]
    gx_out[...] = jnp.concatenate(
        [jnp.dot(x, wxz0[...], preferred_element_type=f32),
         jnp.dot(x, wxh0[...], preferred_element_type=f32)], axis=1)


def _xgate(x, Wxz0, Wxh0):
    return pl.pallas_call(
        _xgate_body,
        grid=(N // BM,),
        in_specs=[
            pl.BlockSpec((BM, F), lambda i: (i, 0)),
            pl.BlockSpec((F, HD), lambda i: (0, 0)),
            pl.BlockSpec((F, HD), lambda i: (0, 0)),
        ],
        out_specs=pl.BlockSpec((BM, 2 * HD), lambda i: (i, 0)),
        out_shape=jax.ShapeDtypeStruct((N, 2 * HD), jnp.float32),
    )(x, Wxz0, Wxh0)


def _dense_body(gxb, ub, degp, wxz1, wxh1, bz, bh,
                w11, b11, w12, b12, w21, b21, w22, b22, wo, bo,
                hn_out, pred_out):
    f32 = jnp.float32
    gx = gxb[...]
    d = degp[0, :, 0:1] + degp[1, :, 0:1]
    dis = jnp.where(d > 0, lax.rsqrt(jnp.maximum(d, 1.0)), 0.0)
    t = dis * jnp.concatenate([ub[0], ub[1]], axis=1)
    zp = (gx[:, :HD]
          + jnp.dot(t, wxz1[...], preferred_element_type=f32) + bz[...])
    hp = (gx[:, HD:]
          + jnp.dot(t, wxh1[...], preferred_element_type=f32) + bh[...])
    z = jax.nn.sigmoid(zp)
    ht = jnp.tanh(hp)
    hn = (1.0 - z) * ht
    hn_out[...] = hn
    y = jax.nn.relu(jnp.dot(hn, w11[...], preferred_element_type=f32) + b11[...])
    y = jax.nn.relu(jnp.dot(y, w12[...], preferred_element_type=f32) + b12[...])
    y = jax.nn.relu(jnp.dot(y, w21[...], preferred_element_type=f32) + b21[...])
    y = jax.nn.relu(jnp.dot(y, w22[...], preferred_element_type=f32) + b22[...])
    pred_out[...] = jnp.dot(y, wo[...], preferred_element_type=f32) + bo[...]


def _row_spec(cols):
    return pl.BlockSpec((BM, cols), lambda i: (i, 0))


def _full_spec(r, c):
    return pl.BlockSpec((r, c), lambda i: (0, 0))


def _dense_chain(gx, u, degp, Wxz1, Wxh1, bz, bh,
                 l1W1, l1b1, l1W2, l1b2, l2W1, l2b1, l2W2, l2b2, outW, outb):
    grid = (N // BM,)
    hn, pred = pl.pallas_call(
        _dense_body,
        grid=grid,
        in_specs=[
            _row_spec(2 * HD),
            pl.BlockSpec((NC, BM, FH), lambda i: (0, i, 0)),
            pl.BlockSpec((NC, BM, FH), lambda i: (0, i, 0)),
            _full_spec(F, HD), _full_spec(F, HD),
            _full_spec(1, HD), _full_spec(1, HD),
            _full_spec(HD, HD), _full_spec(1, HD),
            _full_spec(HD, HD // 2), _full_spec(1, HD // 2),
            _full_spec(HD // 2, HD // 4), _full_spec(1, HD // 4),
            _full_spec(HD // 4, HD // 2), _full_spec(1, HD // 2),
            _full_spec(HD // 2, 1), _full_spec(1, 1),
        ],
        out_specs=[_row_spec(HD), pl.BlockSpec((BM, 1), lambda i: (i, 0))],
        out_shape=[
            jax.ShapeDtypeStruct((N, HD), jnp.float32),
            jax.ShapeDtypeStruct((N, 1), jnp.float32),
        ],
    )(gx, u, degp, Wxz1, Wxh1, bz, bh,
      l1W1, l1b1, l1W2, l1b2, l2W1, l2b1, l2W2, l2b2, outW, outb)
    return hn, pred


def kernel(x, edge_index, Wxz0, Wxz1, bxz, Whz0, Whz1, bhz, Wxr0, Wxr1, bxr,
           Whr0, Whr1, bhr, Wxh0, Wxh1, bxh, Whh0, Whh1, bhh,
           l1W1, l1b1, l1W2, l1b2, l2W1, l2b1, l2W2, l2b2, outW, outb):
    src = edge_index[0]
    dst = edge_index[1]

    pad1 = EPT1 - E // (NC * NS)
    srcd = jnp.concatenate(
        [src.reshape(NC * NS, E // (NC * NS)),
         jnp.full((NC * NS, pad1), N, jnp.int32)], axis=1
    ).reshape(NC, NS, CH1, K1)
    pad = EPT - E // NS
    srcr = jnp.concatenate(
        [src.reshape(NS, E // NS), jnp.zeros((NS, pad), jnp.int32)], axis=1
    ).reshape(NS, CH2, K2)
    srcp = jnp.concatenate([srcr, srcr[:, :4]], axis=1)
    srcf = jnp.stack([srcp, srcp + N]).reshape(-1)
    dst3 = jnp.concatenate(
        [dst.reshape(NS, E // NS), jnp.full((NS, pad), N, jnp.int32)], axis=1
    ).reshape(NS, CH2, K2)

    z128 = jnp.zeros((NP, FH), jnp.float32)
    ones_h = jnp.ones((K1, FH), jnp.float32)

    degp = _deg_call(z128, ones_h, srcd)
    xs2 = _xs_prep(x, degp)
    u = _scat_call(xs2, z128, srcf, dst3)

    bz = (bxz + bhz).reshape(1, HD)
    bh = (bxh + bhh).reshape(1, HD)
    gx = _xgate(x, Wxz0, Wxh0)
    hn, pred = _dense_chain(
        gx, u, degp, Wxz1, Wxh1, bz, bh,
        l1W1, l1b1.reshape(1, HD), l1W2, l1b2.reshape(1, HD // 2),
        l2W1, l2b1.reshape(1, HD // 4), l2W2, l2b2.reshape(1, HD // 2),
        outW, outb.reshape(1, 1))
    return (pred.reshape(-1), hn)


# speedup vs baseline: 1.0363x; 1.0363x over previous
import functools

import jax
import jax.numpy as jnp
from jax import lax
from jax.experimental import pallas as pl
from jax.experimental.pallas import tpu as pltpu
from jax.experimental.pallas import tpu_sc as plsc


N = 10000
E = 160000
F = 256
FH = F // 2
HD = 1024
BM = 1000

NC = 2
NS = 16
NP = 10112
RPT = NP // NS

K2 = 128
EPT = 10240
CH2 = EPT // K2

K1 = 128
EPT1 = 5120
CH1 = EPT1 // K1

_MESH = plsc.VectorSubcoreMesh(core_axis_name="c", subcore_axis_name="s")


def _deg_body(z128, ones_h, srcd, degp_out, deg_sh, ones_v, idx_v):
    c = lax.axis_index("c")
    s = lax.axis_index("s")
    sl = pl.ds(s * RPT, RPT)
    pltpu.sync_copy(z128.at[sl], deg_sh.at[sl])
    pltpu.sync_copy(ones_h, ones_v)
    pltpu.sync_copy(srcd.at[c, s], idx_v)
    plsc.subcore_barrier()

    def body(j, carry):
        pltpu.sync_copy(ones_v, deg_sh.at[idx_v.at[j]], add=True)
        return carry

    lax.fori_loop(0, CH1, body, 0)
    plsc.subcore_barrier()
    pltpu.sync_copy(deg_sh.at[sl], degp_out.at[c, sl])


_deg_call = pl.kernel(
    _deg_body,
    out_type=jax.ShapeDtypeStruct((NC, NP, FH), jnp.float32),
    mesh=_MESH,
    scratch_types=[
        pltpu.VMEM_SHARED((NP, FH), jnp.float32),
        pltpu.VMEM((K1, FH), jnp.float32),
        pltpu.VMEM((CH1, K1), jnp.int32),
    ],
)


def _scat_body(xs2, z128, srcf, dst3, u_out,
               u_sh, dst_v, idx_a, idx_b, rows0, rows1, sem_i, sem_g0, sem_g1):
    c = lax.axis_index("c")
    s = lax.axis_index("s")
    sl = pl.ds(s * RPT, RPT)
    pltpu.sync_copy(z128.at[sl], u_sh.at[sl])
    pltpu.sync_copy(dst3.at[s], dst_v)
    base = (c * NS + s) * ((CH2 + 4) * K2)
    plsc.subcore_barrier()

    pltpu.sync_copy(srcf.at[pl.ds(base, 2 * K2)], idx_a)
    pltpu.async_copy(srcf.at[pl.ds(base + 2 * K2, 2 * K2)], idx_b, sem_i)
    pltpu.async_copy(xs2.at[idx_a.at[pl.ds(0, K2)]], rows0, sem_g0)

    def halfpair(j, i_a, i_b):
        pltpu.async_copy(xs2.at[i_a.at[pl.ds(K2, K2)]], rows1, sem_g1)
        pltpu.make_async_copy(xs2.at[i_a.at[pl.ds(0, K2)]], rows0, sem_g0).wait()
        pltpu.sync_copy(rows0, u_sh.at[dst_v.at[j]], add=True)
        pltpu.make_async_copy(srcf.at[pl.ds(base, 2 * K2)], i_b, sem_i).wait()
        pltpu.async_copy(xs2.at[i_b.at[pl.ds(0, K2)]], rows0, sem_g0)
        pltpu.make_async_copy(xs2.at[i_a.at[pl.ds(K2, K2)]], rows1, sem_g1).wait()
        pltpu.sync_copy(rows1, u_sh.at[dst_v.at[j + 1]], add=True)
        pltpu.async_copy(srcf.at[pl.ds(base + (j + 4) * K2, 2 * K2)], i_a, sem_i)

    def q_body(q, carry):
        j = 4 * q
        halfpair(j, idx_a, idx_b)
        halfpair(j + 2, idx_b, idx_a)
        return carry

    lax.fori_loop(0, CH2 // 4, q_body, 0)
    pltpu.make_async_copy(srcf.at[pl.ds(base, 2 * K2)], idx_b, sem_i).wait()
    pltpu.make_async_copy(xs2.at[idx_a.at[pl.ds(0, K2)]], rows0, sem_g0).wait()
    plsc.subcore_barrier()
    pltpu.sync_copy(u_sh.at[sl], u_out.at[c, sl])


_scat_call = pl.kernel(
    _scat_body,
    out_type=jax.ShapeDtypeStruct((NC, NP, FH), jnp.float32),
    mesh=_MESH,
    scratch_types=[
        pltpu.VMEM_SHARED((NP, FH), jnp.float32),
        pltpu.VMEM((CH2, K2), jnp.int32),
        pltpu.VMEM((2 * K2,), jnp.int32),
        pltpu.VMEM((2 * K2,), jnp.int32),
        pltpu.VMEM((K2, FH), jnp.float32),
        pltpu.VMEM((K2, FH), jnp.float32),
        pltpu.SemaphoreType.DMA,
        pltpu.SemaphoreType.DMA,
        pltpu.SemaphoreType.DMA,
    ],
)


def _xs_body(x_ref, degp_ref, xs2_ref):
    d = degp_ref[0, :, 0:1] + degp_ref[1, :, 0:1]
    dis = jnp.where(d > 0, lax.rsqrt(jnp.maximum(d, 1.0)), 0.0)
    xs2_ref[...] = (-dis) * x_ref[...]


def _xs_prep(x, degp):
    nb = N // BM
    return pl.pallas_call(
        _xs_body,
        grid=(2, nb),
        in_specs=[
            pl.BlockSpec((BM, FH), lambda h, i: (i, h)),
            pl.BlockSpec((NC, BM, FH), lambda h, i: (0, i, 0)),
        ],
        out_specs=pl.BlockSpec((BM, FH), lambda h, i: (h * (N // BM) + i, 0)),
        out_shape=jax.ShapeDtypeStruct((2 * N, FH), jnp.float32),
    )(x, degp)


def _dense_body(xb, ub, degp, wxz0, wxz1, wxh0, wxh1, bz, bh,
                w11, b11, w12, b12, w21, b21, w22, b22, wo, bo,
                hn_out, pred_out):
    f32 = jnp.float32
    x = xb[...]
    d = degp[0, :, 0:1] + degp[1, :, 0:1]
    dis = jnp.where(d > 0, lax.rsqrt(jnp.maximum(d, 1.0)), 0.0)
    t = dis * jnp.concatenate([ub[0], ub[1]], axis=1)
    zp = (jnp.dot(x, wxz0[...], preferred_element_type=f32)
          + jnp.dot(t, wxz1[...], preferred_element_type=f32) + bz[...])
    hp = (jnp.dot(x, wxh0[...], preferred_element_type=f32)
          + jnp.dot(t, wxh1[...], preferred_element_type=f32) + bh[...])
    z = jax.nn.sigmoid(zp)
    ht = jnp.tanh(hp)
    hn = (1.0 - z) * ht
    hn_out[...] = hn
    y = jax.nn.relu(jnp.dot(hn, w11[...], preferred_element_type=f32) + b11[...])
    y = jax.nn.relu(jnp.dot(y, w12[...], preferred_element_type=f32) + b12[...])
    y = jax.nn.relu(jnp.dot(y, w21[...], preferred_element_type=f32) + b21[...])
    y = jax.nn.relu(jnp.dot(y, w22[...], preferred_element_type=f32) + b22[...])
    pred_out[...] = jnp.dot(y, wo[...], preferred_element_type=f32) + bo[...]


def _row_spec(cols):
    return pl.BlockSpec((BM, cols), lambda i: (i, 0))


def _full_spec(r, c):
    return pl.BlockSpec((r, c), lambda i: (0, 0))


def _dense_chain(x, u, degp, Wxz0, Wxz1, Wxh0, Wxh1, bz, bh,
                 l1W1, l1b1, l1W2, l1b2, l2W1, l2b1, l2W2, l2b2, outW, outb):
    grid = (N // BM,)
    hn, pred = pl.pallas_call(
        _dense_body,
        grid=grid,
        in_specs=[
            _row_spec(F),
            pl.BlockSpec((NC, BM, FH), lambda i: (0, i, 0)),
            pl.BlockSpec((NC, BM, FH), lambda i: (0, i, 0)),
            _full_spec(F, HD), _full_spec(F, HD), _full_spec(F, HD), _full_spec(F, HD),
            _full_spec(1, HD), _full_spec(1, HD),
            _full_spec(HD, HD), _full_spec(1, HD),
            _full_spec(HD, HD // 2), _full_spec(1, HD // 2),
            _full_spec(HD // 2, HD // 4), _full_spec(1, HD // 4),
            _full_spec(HD // 4, HD // 2), _full_spec(1, HD // 2),
            _full_spec(HD // 2, 1), _full_spec(1, 1),
        ],
        out_specs=[_row_spec(HD), pl.BlockSpec((BM, 1), lambda i: (i, 0))],
        out_shape=[
            jax.ShapeDtypeStruct((N, HD), jnp.float32),
            jax.ShapeDtypeStruct((N, 1), jnp.float32),
        ],
    )(x, u, degp, Wxz0, Wxz1, Wxh0, Wxh1, bz, bh,
      l1W1, l1b1, l1W2, l1b2, l2W1, l2b1, l2W2, l2b2, outW, outb)
    return hn, pred


def kernel(x, edge_index, Wxz0, Wxz1, bxz, Whz0, Whz1, bhz, Wxr0, Wxr1, bxr,
           Whr0, Whr1, bhr, Wxh0, Wxh1, bxh, Whh0, Whh1, bhh,
           l1W1, l1b1, l1W2, l1b2, l2W1, l2b1, l2W2, l2b2, outW, outb):
    src = edge_index[0]
    dst = edge_index[1]

    pad1 = EPT1 - E // (NC * NS)
    srcd = jnp.concatenate(
        [src.reshape(NC * NS, E // (NC * NS)),
         jnp.full((NC * NS, pad1), N, jnp.int32)], axis=1
    ).reshape(NC, NS, CH1, K1)
    pad = EPT - E // NS
    srcr = jnp.concatenate(
        [src.reshape(NS, E // NS), jnp.zeros((NS, pad), jnp.int32)], axis=1
    ).reshape(NS, CH2, K2)
    srcp = jnp.concatenate([srcr, srcr[:, :4]], axis=1)
    srcf = jnp.stack([srcp, srcp + N]).reshape(-1)
    dst3 = jnp.concatenate(
        [dst.reshape(NS, E // NS), jnp.full((NS, pad), N, jnp.int32)], axis=1
    ).reshape(NS, CH2, K2)

    z128 = jnp.zeros((NP, FH), jnp.float32)
    ones_h = jnp.ones((K1, FH), jnp.float32)

    degp = _deg_call(z128, ones_h, srcd)
    xs2 = _xs_prep(x, degp)
    u = _scat_call(xs2, z128, srcf, dst3)

    bz = (bxz + bhz).reshape(1, HD)
    bh = (bxh + bhh).reshape(1, HD)
    hn, pred = _dense_chain(
        x, u, degp, Wxz0, Wxz1, Wxh0, Wxh1, bz, bh,
        l1W1, l1b1.reshape(1, HD), l1W2, l1b2.reshape(1, HD // 2),
        l2W1, l2b1.reshape(1, HD // 4), l2W2, l2b2.reshape(1, HD // 2),
        outW, outb.reshape(1, 1))
    return (pred.reshape(-1), hn)


# speedup vs baseline: 1.0519x; 1.0150x over previous
import functools

import jax
import jax.numpy as jnp
from jax import lax
from jax.experimental import pallas as pl
from jax.experimental.pallas import tpu as pltpu
from jax.experimental.pallas import tpu_sc as plsc


N = 10000
E = 160000
F = 256
FH = F // 2
HD = 1024
BM = 2000

NC = 2
NS = 16
NP = 10112
RPT = NP // NS

K2 = 128
EPT = 10240
CH2 = EPT // K2

K1 = 128
EPT1 = 5120
CH1 = EPT1 // K1

_MESH = plsc.VectorSubcoreMesh(core_axis_name="c", subcore_axis_name="s")


def _deg_body(z128, ones_h, srcd, degp_out, deg_sh, ones_v, idx_v):
    c = lax.axis_index("c")
    s = lax.axis_index("s")
    sl = pl.ds(s * RPT, RPT)
    pltpu.sync_copy(z128.at[sl], deg_sh.at[sl])
    pltpu.sync_copy(ones_h, ones_v)
    pltpu.sync_copy(srcd.at[c, s], idx_v)
    plsc.subcore_barrier()

    def body(j, carry):
        pltpu.sync_copy(ones_v, deg_sh.at[idx_v.at[j]], add=True)
        return carry

    lax.fori_loop(0, CH1, body, 0)
    plsc.subcore_barrier()
    pltpu.sync_copy(deg_sh.at[sl], degp_out.at[c, sl])


_deg_call = pl.kernel(
    _deg_body,
    out_type=jax.ShapeDtypeStruct((NC, NP, FH), jnp.float32),
    mesh=_MESH,
    scratch_types=[
        pltpu.VMEM_SHARED((NP, FH), jnp.float32),
        pltpu.VMEM((K1, FH), jnp.float32),
        pltpu.VMEM((CH1, K1), jnp.int32),
    ],
)


def _scat_body(xs2, z128, srcf, dst3, u_out,
               u_sh, dst_v, idx_a, idx_b, rows0, rows1, sem_i, sem_g0, sem_g1):
    c = lax.axis_index("c")
    s = lax.axis_index("s")
    sl = pl.ds(s * RPT, RPT)
    pltpu.sync_copy(z128.at[sl], u_sh.at[sl])
    pltpu.sync_copy(dst3.at[s], dst_v)
    base = (c * NS + s) * ((CH2 + 4) * K2)
    plsc.subcore_barrier()

    pltpu.sync_copy(srcf.at[pl.ds(base, 2 * K2)], idx_a)
    pltpu.async_copy(srcf.at[pl.ds(base + 2 * K2, 2 * K2)], idx_b, sem_i)
    pltpu.async_copy(xs2.at[idx_a.at[pl.ds(0, K2)]], rows0, sem_g0)

    def halfpair(j, i_a, i_b):
        pltpu.async_copy(xs2.at[i_a.at[pl.ds(K2, K2)]], rows1, sem_g1)
        pltpu.make_async_copy(xs2.at[i_a.at[pl.ds(0, K2)]], rows0, sem_g0).wait()
        pltpu.sync_copy(rows0, u_sh.at[dst_v.at[j]], add=True)
        pltpu.make_async_copy(srcf.at[pl.ds(base, 2 * K2)], i_b, sem_i).wait()
        pltpu.async_copy(xs2.at[i_b.at[pl.ds(0, K2)]], rows0, sem_g0)
        pltpu.make_async_copy(xs2.at[i_a.at[pl.ds(K2, K2)]], rows1, sem_g1).wait()
        pltpu.sync_copy(rows1, u_sh.at[dst_v.at[j + 1]], add=True)
        pltpu.async_copy(srcf.at[pl.ds(base + (j + 4) * K2, 2 * K2)], i_a, sem_i)

    def q_body(q, carry):
        j = 4 * q
        halfpair(j, idx_a, idx_b)
        halfpair(j + 2, idx_b, idx_a)
        return carry

    lax.fori_loop(0, CH2 // 4, q_body, 0)
    pltpu.make_async_copy(srcf.at[pl.ds(base, 2 * K2)], idx_b, sem_i).wait()
    pltpu.make_async_copy(xs2.at[idx_a.at[pl.ds(0, K2)]], rows0, sem_g0).wait()
    plsc.subcore_barrier()
    pltpu.sync_copy(u_sh.at[sl], u_out.at[c, sl])


_scat_call = pl.kernel(
    _scat_body,
    out_type=jax.ShapeDtypeStruct((NC, NP, FH), jnp.float32),
    mesh=_MESH,
    scratch_types=[
        pltpu.VMEM_SHARED((NP, FH), jnp.float32),
        pltpu.VMEM((CH2, K2), jnp.int32),
        pltpu.VMEM((2 * K2,), jnp.int32),
        pltpu.VMEM((2 * K2,), jnp.int32),
        pltpu.VMEM((K2, FH), jnp.float32),
        pltpu.VMEM((K2, FH), jnp.float32),
        pltpu.SemaphoreType.DMA,
        pltpu.SemaphoreType.DMA,
        pltpu.SemaphoreType.DMA,
    ],
)


def _xs_body(x_ref, degp_ref, xs2_ref):
    d = degp_ref[0, :, 0:1] + degp_ref[1, :, 0:1]
    dis = jnp.where(d > 0, lax.rsqrt(jnp.maximum(d, 1.0)), 0.0)
    xs2_ref[...] = (-dis) * x_ref[...]


def _xs_prep(x, degp):
    nb = N // BM
    return pl.pallas_call(
        _xs_body,
        grid=(2, nb),
        in_specs=[
            pl.BlockSpec((BM, FH), lambda h, i: (i, h)),
            pl.BlockSpec((NC, BM, FH), lambda h, i: (0, i, 0)),
        ],
        out_specs=pl.BlockSpec((BM, FH), lambda h, i: (h * (N // BM) + i, 0)),
        out_shape=jax.ShapeDtypeStruct((2 * N, FH), jnp.float32),
    )(x, degp)


def _dense_body(xb, ub, degp, wxz0, wxz1, wxh0, wxh1, bz, bh,
                w11, b11, w12, b12, w21, b21, w22, b22, wo, bo,
                hn_out, pred_out):
    f32 = jnp.float32
    x = xb[...]
    d = degp[0, :, 0:1] + degp[1, :, 0:1]
    dis = jnp.where(d > 0, lax.rsqrt(jnp.maximum(d, 1.0)), 0.0)
    t = dis * jnp.concatenate([ub[0], ub[1]], axis=1)
    zp = (jnp.dot(x, wxz0[...], preferred_element_type=f32)
          + jnp.dot(t, wxz1[...], preferred_element_type=f32) + bz[...])
    hp = (jnp.dot(x, wxh0[...], preferred_element_type=f32)
          + jnp.dot(t, wxh1[...], preferred_element_type=f32) + bh[...])
    z = jax.nn.sigmoid(zp)
    ht = jnp.tanh(hp)
    hn = (1.0 - z) * ht
    hn_out[...] = hn
    y = jax.nn.relu(jnp.dot(hn, w11[...], preferred_element_type=f32) + b11[...])
    y = jax.nn.relu(jnp.dot(y, w12[...], preferred_element_type=f32) + b12[...])
    y = jax.nn.relu(jnp.dot(y, w21[...], preferred_element_type=f32) + b21[...])
    y = jax.nn.relu(jnp.dot(y, w22[...], preferred_element_type=f32) + b22[...])
    pred_out[...] = jnp.dot(y, wo[...], preferred_element_type=f32) + bo[...]


def _row_spec(cols):
    return pl.BlockSpec((BM, cols), lambda i: (i, 0))


def _full_spec(r, c):
    return pl.BlockSpec((r, c), lambda i: (0, 0))


def _dense_chain(x, u, degp, Wxz0, Wxz1, Wxh0, Wxh1, bz, bh,
                 l1W1, l1b1, l1W2, l1b2, l2W1, l2b1, l2W2, l2b2, outW, outb):
    grid = (N // BM,)
    hn, pred = pl.pallas_call(
        _dense_body,
        grid=grid,
        in_specs=[
            _row_spec(F),
            pl.BlockSpec((NC, BM, FH), lambda i: (0, i, 0)),
            pl.BlockSpec((NC, BM, FH), lambda i: (0, i, 0)),
            _full_spec(F, HD), _full_spec(F, HD), _full_spec(F, HD), _full_spec(F, HD),
            _full_spec(1, HD), _full_spec(1, HD),
            _full_spec(HD, HD), _full_spec(1, HD),
            _full_spec(HD, HD // 2), _full_spec(1, HD // 2),
            _full_spec(HD // 2, HD // 4), _full_spec(1, HD // 4),
            _full_spec(HD // 4, HD // 2), _full_spec(1, HD // 2),
            _full_spec(HD // 2, 1), _full_spec(1, 1),
        ],
        out_specs=[_row_spec(HD), pl.BlockSpec((BM, 1), lambda i: (i, 0))],
        out_shape=[
            jax.ShapeDtypeStruct((N, HD), jnp.float32),
            jax.ShapeDtypeStruct((N, 1), jnp.float32),
        ],
    )(x, u, degp, Wxz0, Wxz1, Wxh0, Wxh1, bz, bh,
      l1W1, l1b1, l1W2, l1b2, l2W1, l2b1, l2W2, l2b2, outW, outb)
    return hn, pred


def kernel(x, edge_index, Wxz0, Wxz1, bxz, Whz0, Whz1, bhz, Wxr0, Wxr1, bxr,
           Whr0, Whr1, bhr, Wxh0, Wxh1, bxh, Whh0, Whh1, bhh,
           l1W1, l1b1, l1W2, l1b2, l2W1, l2b1, l2W2, l2b2, outW, outb):
    src = edge_index[0]
    dst = edge_index[1]

    pad1 = EPT1 - E // (NC * NS)
    srcd = jnp.concatenate(
        [src.reshape(NC * NS, E // (NC * NS)),
         jnp.full((NC * NS, pad1), N, jnp.int32)], axis=1
    ).reshape(NC, NS, CH1, K1)
    pad = EPT - E // NS
    srcr = jnp.concatenate(
        [src.reshape(NS, E // NS), jnp.zeros((NS, pad), jnp.int32)], axis=1
    ).reshape(NS, CH2, K2)
    srcp = jnp.concatenate([srcr, srcr[:, :4]], axis=1)
    srcf = jnp.stack([srcp, srcp + N]).reshape(-1)
    dst3 = jnp.concatenate(
        [dst.reshape(NS, E // NS), jnp.full((NS, pad), N, jnp.int32)], axis=1
    ).reshape(NS, CH2, K2)

    z128 = jnp.zeros((NP, FH), jnp.float32)
    ones_h = jnp.ones((K1, FH), jnp.float32)

    degp = _deg_call(z128, ones_h, srcd)
    xs2 = _xs_prep(x, degp)
    u = _scat_call(xs2, z128, srcf, dst3)

    bz = (bxz + bhz).reshape(1, HD)
    bh = (bxh + bhh).reshape(1, HD)
    hn, pred = _dense_chain(
        x, u, degp, Wxz0, Wxz1, Wxh0, Wxh1, bz, bh,
        l1W1, l1b1.reshape(1, HD), l1W2, l1b2.reshape(1, HD // 2),
        l2W1, l2b1.reshape(1, HD // 4), l2W2, l2b2.reshape(1, HD // 2),
        outW, outb.reshape(1, 1))
    return (pred.reshape(-1), hn)
